# baseline (speedup 1.0000x reference)
import functools

import jax
import jax.numpy as jnp
from jax import lax
from jax.experimental import pallas as pl
from jax.experimental.pallas import tpu as pltpu
from jax.experimental.pallas import tpu_sc as plsc


def _splat(x):
    return jnp.full((16,), x, dtype=jnp.int32)


def _build_sc_hash(n_tokens, seq, mod):
    info = plsc.get_sparse_core_info()
    nc, ns = info.num_cores, info.num_subcores
    nw = nc * ns
    chunk = n_tokens // nw
    nvec = chunk // 16
    n_streams = chunk // 128
    mesh = plsc.VectorSubcoreMesh(core_axis_name="c", subcore_axis_name="s")

    @functools.partial(
        pl.kernel,
        mesh=mesh,
        compiler_params=pltpu.CompilerParams(use_tc_tiling_on_sc=True),
        out_type=jax.ShapeDtypeStruct((n_tokens,), jnp.int32),
        scratch_types=[
            pltpu.VMEM((chunk,), jnp.int32),
            pltpu.VMEM((chunk,), jnp.int32),
            pltpu.VMEM((n_streams, 128), jnp.int32),
            pltpu.VMEM((chunk,), jnp.int32),
            pltpu.SemaphoreType.DMA,
        ],
    )
    def hash_kernel(tok_hbm, hall_hbm, tok_v, prev_v, pidx_v, hid_v, sem):
        wid = lax.axis_index("s") * nc + lax.axis_index("c")
        base = wid * chunk
        lane = lax.iota(jnp.int32, 16)
        for j in range(nvec):
            pidx = jnp.maximum(base + (j * 16 - 1) + lane, 0)
            pidx_v[j // 8, pl.ds((j % 8) * 16, 16)] = pidx
        tok_cp = pltpu.async_copy(tok_hbm.at[pl.ds(base, chunk)], tok_v, sem)
        prev_cps = [
            pltpu.async_copy(
                tok_hbm.at[pidx_v.at[i]], prev_v.at[pl.ds(i * 128, 128)], sem
            )
            for i in range(n_streams)
        ]
        tok_cp.wait()
        for c in prev_cps:
            c.wait()
        row_start = 1 - jnp.minimum(base % seq, 1)
        lane0 = jnp.maximum(1 - lane, 0)
        for j in range(nvec):
            cur = tok_v[pl.ds(j * 16, 16)]
            prev = prev_v[pl.ds(j * 16, 16)]
            h = (36313 * cur ^ 27191 * prev) % mod
            if j == 0:
                sel = lane0 * row_start
                h = h + sel * (mod - h)
            hid_v[pl.ds(j * 16, 16)] = h
        pltpu.sync_copy(hid_v, hall_hbm.at[pl.ds(base, chunk)])

    return hash_kernel


def _build_sc_sweep(n_tokens, dim, vocab):
    info = plsc.get_sparse_core_info()
    nc, ns = info.num_cores, info.num_subcores
    nw = nc * ns
    q_full = vocab // 128
    tail0 = q_full * 128
    qw = -(-q_full // nw)
    rt = 2
    nrounds = -(-qw // rt)
    tcap = 624
    ntvec = n_tokens // 16
    mesh = plsc.VectorSubcoreMesh(core_axis_name="c", subcore_axis_name="s")

    @functools.partial(
        pl.kernel,
        mesh=mesh,
        compiler_params=pltpu.CompilerParams(
            use_tc_tiling_on_sc=True, needs_layout_passes=False
        ),
        out_type=jax.ShapeDtypeStruct((n_tokens, dim), jnp.float32),
        scratch_types=[
            pltpu.VMEM((n_tokens,), jnp.int32),
            pltpu.VMEM((tcap,), jnp.int32),
            pltpu.VMEM((tcap,), jnp.int32),
            pltpu.VMEM((dim, rt * 128), jnp.float32),
            pltpu.VMEM((dim, rt * 128), jnp.float32),
            pltpu.VMEM((tcap, dim), jnp.float32),
            pltpu.SemaphoreType.DMA,
            pltpu.SemaphoreType.DMA,
        ],
    )
    def sweep_kernel(hall_hbm, table_hbm, tail_hbm, out_hbm, hall_v, li_v,
                     lh_v, tile_a, tile_b, rloc_v, sem, osem):
        bufs = (tile_a, tile_b)
        wid = lax.axis_index("s") * nc + lax.axis_index("c")
        qlo = wid * qw
        qhi = jnp.minimum(qlo + qw, q_full) + jnp.minimum(
            jnp.maximum(wid - (nw - 2), 0), 1
        )
        lane = lax.iota(jnp.int32, 16)
        pltpu.sync_copy(hall_hbm, hall_v)

        def count_step(v, cnt):
            off = pl.multiple_of(v * 16, 16)
            qv = hall_v[pl.ds(off, 16)] >> 7
            m = jnp.logical_and(qv >= qlo, qv < qhi)
            return cnt + plsc.all_reduce_population_count(m)[0]

        cnt = lax.fori_loop(0, ntvec, count_step, 0)
        npass = (cnt + tcap - 1) // tcap

        def col_of(r):
            lo = jnp.minimum(qlo + r * rt, q_full - rt)
            return lo, pl.multiple_of(lo * 128, 128)

        def one_pass(ci, _):
            cbase = ci * tcap

            def compact(v, c2):
                off = pl.multiple_of(v * 16, 16)
                hv = hall_v[pl.ds(off, 16)]
                qv = hv >> 7
                m = jnp.logical_and(qv >= qlo, qv < qhi)
                mi = m.astype(jnp.int32)
                pos = c2 + plsc.cumsum(mi) - mi
                ms = jnp.logical_and(m, pos >= cbase)
                ms = jnp.logical_and(ms, pos < cbase + tcap)
                plsc.store_scatter(li_v, [pos - cbase], off + lane, mask=ms)
                plsc.store_scatter(lh_v, [pos - cbase], hv, mask=ms)
                return c2 + plsc.all_reduce_population_count(m)[0]

            seen = lax.fori_loop(0, ntvec, compact, 0)
            nthis = jnp.minimum(seen - cbase, tcap)
            nvloc = (nthis + 15) >> 4

            def extract_from(vl, carry, tile, lo_c, is_tail):
                issued = carry
                off = pl.multiple_of(vl * 16, 16)
                hv = lh_v[pl.ds(off, 16)]
                qv = hv >> 7
                gpos = off + lane
                span = 1 if is_tail else rt
                m = (qv - lo_c).astype(jnp.uint32) < jnp.uint32(span)
                m = jnp.logical_and(m, gpos < nthis)
                mi0 = m.astype(jnp.int32)

                def wcond(carry2):
                    mi, _ = carry2
                    return plsc.all_reduce_population_count(mi == 1)[0] > 0

                def wbody(carry2):
                    mi, iss = carry2
                    l = plsc.all_reduce_ffs(mi == 1)[0]
                    p = off + l
                    h_s = plsc.load_gather(lh_v, [_splat(p)])[0]
                    i_s = plsc.load_gather(li_v, [_splat(p)])[0]
                    if is_tail:
                        pltpu.async_copy(
                            tail_hbm.at[h_s - tail0], rloc_v.at[p], sem
                        ).wait()
                    else:
                        c = h_s - lo_c * 128
                        for k in range(dim // 16):
                            vals = plsc.load_gather(
                                tile, [lane + 16 * k, _splat(c)]
                            )
                            rloc_v[p, pl.ds(16 * k, 16)] = vals
                    pltpu.async_copy(rloc_v.at[p], out_hbm.at[i_s], osem)
                    mi2 = mi * (lane != l).astype(jnp.int32)
                    return (mi2, iss + 1)

                _, issued2 = lax.while_loop(wcond, wbody, (mi0, issued))
                return issued2

            _, c0 = col_of(0)
            pltpu.async_copy(table_hbm.at[:, pl.ds(c0, rt * 128)], tile_a, sem)

            def round_pair(i, issued):
                r0 = 2 * i
                _, cb = col_of(r0 + 1)
                pltpu.async_copy(table_hbm.at[:, pl.ds(cb, rt * 128)],
                                 tile_b, sem)
                pltpu.make_async_copy(
                    table_hbm.at[:, pl.ds(c0, rt * 128)], tile_a, sem
                ).wait()
                lo_a, _ = col_of(r0)
                issued = lax.fori_loop(
                    0, nvloc,
                    functools.partial(
                        extract_from, tile=tile_a, lo_c=lo_a, is_tail=False
                    ),
                    issued,
                )
                _, ca = col_of(r0 + 2)
                pltpu.async_copy(table_hbm.at[:, pl.ds(ca, rt * 128)],
                                 tile_a, sem)
                pltpu.make_async_copy(
                    table_hbm.at[:, pl.ds(c0, rt * 128)], tile_b, sem
                ).wait()
                lo_b, _ = col_of(r0 + 1)
                issued = lax.fori_loop(
                    0, nvloc,
                    functools.partial(
                        extract_from, tile=tile_b, lo_c=lo_b, is_tail=False
                    ),
                    issued,
                )
                return issued

            issued = lax.fori_loop(0, (nrounds + 1) // 2, round_pair, 0)
            pltpu.make_async_copy(
                table_hbm.at[:, pl.ds(c0, rt * 128)], tile_a, sem
            ).wait()
            issued = lax.fori_loop(
                0, nvloc,
                functools.partial(
                    extract_from, tile=tile_a, lo_c=q_full, is_tail=True
                ),
                issued,
            )

            def drain(i, _):
                pltpu.make_async_copy(tail_hbm.at[0], rloc_v.at[0], osem).wait()
                return 0

            lax.fori_loop(0, issued, drain, 0)
            return 0

        lax.fori_loop(0, npass, one_pass, 0)

    return sweep_kernel


def _tc_project(h, proj_t, scale, bm=1024):
    n, dim = h.shape
    dm = proj_t.shape[1]

    def body(s_ref, h_ref, p_ref, o_ref):
        o_ref[...] = (
            jnp.dot(
                h_ref[...].astype(jnp.bfloat16),
                p_ref[...].astype(jnp.bfloat16),
                preferred_element_type=jnp.float32,
            )
            * s_ref[0]
        )

    return pl.pallas_call(
        body,
        grid=(n // bm,),
        in_specs=[
            pl.BlockSpec(memory_space=pltpu.SMEM),
            pl.BlockSpec((bm, dim), lambda i: (i, 0)),
            pl.BlockSpec((dim, dm), lambda i: (0, 0)),
        ],
        out_specs=pl.BlockSpec((bm, dm), lambda i: (i, 0)),
        out_shape=jax.ShapeDtypeStruct((n, dm), jnp.float32),
    )(scale.reshape(1), h, proj_t)


def kernel(token_ids, embed_w, proj_w, scale):
    b, s = token_ids.shape
    vocab, dim = embed_w.shape
    dm = proj_w.shape[0]
    tok = token_ids.reshape(-1).astype(jnp.int32)
    n = b * s
    h_all = _build_sc_hash(n, s, vocab - 1)(tok)
    tail0 = (vocab // 128) * 128
    tail = lax.slice(embed_w, (tail0, 0), (vocab, dim))
    rows = _build_sc_sweep(n, dim, vocab)(h_all, embed_w.T, tail)
    out = _tc_project(rows, proj_w.T, scale)
    return out.reshape(b, s, dm)
